# baseline (speedup 1.0000x reference)
import functools

import jax
import jax.numpy as jnp
from jax import lax
from jax.experimental import pallas as pl
from jax.experimental.pallas import tpu as pltpu
from jax.experimental.pallas import tpu_sc as plsc

_EPS = 1e-12
_LANES = 16


def _rsqrt_newton(v):
    i = lax.bitcast_convert_type(v, jnp.int32)
    i = jnp.full(v.shape, 0x5F3759DF, jnp.int32) - lax.shift_right_logical(i, 1)
    y = lax.bitcast_convert_type(i, jnp.float32)
    h = 0.5 * v
    for _ in range(3):
        y = y * (1.5 - h * y * y)
    return y


def _splat_sum(v):
    dnums = lax.GatherDimensionNumbers(
        offset_dims=(), collapsed_slice_dims=(0,), start_index_map=(0,))
    lane = lax.iota(jnp.int32, _LANES)
    for sh in (8, 4, 2, 1):
        idx = lax.bitwise_and(lane + sh, _LANES - 1)
        perm = lax.gather(v, idx[:, None], dimension_numbers=dnums,
                          slice_sizes=(1,),
                          mode=lax.GatherScatterMode.PROMISE_IN_BOUNDS)
        v = v + perm
    return v


def _make_sc_kernel(n_rows, seq, hidden, chunk):
    info = plsc.get_sparse_core_info()
    nc, ns = info.num_cores, info.num_subcores
    nw = nc * ns
    batch = n_rows // seq
    span = seq // nw
    n_chunks = span // chunk
    n_vec = hidden // _LANES

    mesh = plsc.VectorSubcoreMesh(core_axis_name="c", subcore_axis_name="s")

    n_items = n_chunks * batch

    @functools.partial(
        pl.kernel,
        mesh=mesh,
        out_type=jax.ShapeDtypeStruct((n_rows, hidden), jnp.float32),
        scratch_types=[
            pltpu.VMEM((chunk, hidden), jnp.float32),
            pltpu.VMEM((chunk, hidden), jnp.float32),
            pltpu.VMEM((chunk, hidden), jnp.float32),
            pltpu.VMEM((hidden,), jnp.float32),
            pltpu.VMEM((hidden,), jnp.float32),
            pltpu.VMEM((chunk, _LANES), jnp.float32),
            pltpu.VMEM((chunk, _LANES), jnp.float32),
            pltpu.SemaphoreType.DMA,
            pltpu.SemaphoreType.DMA,
            pltpu.SemaphoreType.DMA,
            pltpu.SemaphoreType.DMA,
        ],
    )
    def sc_kernel(in_hbm, pos_hbm, gam_hbm, bet_hbm, out_hbm,
                  x0, x1, pos_v, gam_v, bet_v, mean_v, rs_v,
                  si0, si1, so0, so1):
        wid = lax.axis_index("s") * nc + lax.axis_index("c")
        s0 = wid * span
        xs = (x0, x1)
        sin = (si0, si1)
        sout = (so0, so1)
        pltpu.sync_copy(gam_hbm, gam_v)
        pltpu.sync_copy(bet_hbm, bet_v)

        def seq_base(item):
            return s0 + lax.shift_right_logical(item, 2) * chunk

        def io_row(item):
            return lax.bitwise_and(item, 3) * seq + seq_base(item)

        def in_cp(sl, item):
            return pltpu.make_async_copy(
                in_hbm.at[pl.ds(io_row(item), chunk)], xs[sl], sin[sl])

        def out_cp(sl, item):
            return pltpu.make_async_copy(
                xs[sl], out_hbm.at[pl.ds(io_row(item), chunk)], sout[sl])

        def compute(x_v):
            @plsc.parallel_loop(0, chunk, unroll=4)
            def _(t):
                a1 = [jnp.zeros((_LANES,), jnp.float32) for _ in range(2)]
                a2 = [jnp.zeros((_LANES,), jnp.float32) for _ in range(2)]
                for i in range(n_vec):
                    a = i & 1
                    sl = pl.ds(i * _LANES, _LANES)
                    xv = x_v[t, sl] + pos_v[t, sl]
                    x_v[t, sl] = xv
                    a1[a] = a1[a] + xv
                    a2[a] = a2[a] + xv * xv
                s1 = a1[0] + a1[1]
                s2 = a2[0] + a2[1]
                mean = _splat_sum(s1) * (1.0 / hidden)
                var = _splat_sum(s2) * (1.0 / hidden) - mean * mean
                mean_v[t] = mean
                rs_v[t] = _rsqrt_newton(var + _EPS)

            grp = 24
            for g0 in range(0, n_vec, grp):
                gs = [gam_v[pl.ds(i * _LANES, _LANES)]
                      for i in range(g0, g0 + grp)]
                bs = [bet_v[pl.ds(i * _LANES, _LANES)]
                      for i in range(g0, g0 + grp)]

                @plsc.parallel_loop(0, chunk, unroll=2)
                def _(t, g0=g0, gs=gs, bs=bs):
                    r = rs_v[t]
                    mr = mean_v[t] * r
                    for k in range(grp):
                        sl = pl.ds((g0 + k) * _LANES, _LANES)
                        x_v[t, sl] = (x_v[t, sl] * r - mr) * gs[k] + bs[k]

        in_cp(0, 0).start()
        in_cp(1, 1).start()

        def pair_body(p, _):
            i0 = 2 * p
            for sl in range(2):
                item = i0 + sl
                if sl == 0:
                    @pl.when(lax.bitwise_and(item, 3) == 0)
                    def _():
                        pltpu.sync_copy(
                            pos_hbm.at[pl.ds(seq_base(item), chunk)], pos_v)
                in_cp(sl, item).wait()
                compute(xs[sl])
                out_cp(sl, item).start()
            for sl in range(2):
                item = i0 + sl
                @pl.when(item + 2 < n_items)
                def _():
                    out_cp(sl, item).wait()
                    in_cp(sl, item + 2).start()
            return 0

        lax.fori_loop(0, n_items // 2, pair_body, 0)
        out_cp(0, n_items - 2).wait()
        out_cp(1, n_items - 1).wait()

    return sc_kernel


def kernel(inputs_embeds, position_table, ln_gamma, ln_beta):
    batch, seq, hidden = inputs_embeds.shape
    flat = inputs_embeds.reshape(batch * seq, hidden)
    pos = position_table[:seq]
    fn = _make_sc_kernel(batch * seq, seq, hidden, chunk=32)
    out = fn(flat, pos, ln_gamma, ln_beta)
    return out.reshape(batch, seq, hidden)


# speedup vs baseline: 1.2372x; 1.2372x over previous
import functools

import jax
import jax.numpy as jnp
from jax import lax
from jax.experimental import pallas as pl
from jax.experimental.pallas import tpu as pltpu
from jax.experimental.pallas import tpu_sc as plsc

_EPS = 1e-12
_LANES = 16


def _rsqrt_newton(v):
    i = lax.bitcast_convert_type(v, jnp.int32)
    i = jnp.full(v.shape, 0x5F3759DF, jnp.int32) - lax.shift_right_logical(i, 1)
    y = lax.bitcast_convert_type(i, jnp.float32)
    h = 0.5 * v
    for _ in range(3):
        y = y * (1.5 - h * y * y)
    return y


def _splat_sum(v):
    dnums = lax.GatherDimensionNumbers(
        offset_dims=(), collapsed_slice_dims=(0,), start_index_map=(0,))
    lane = lax.iota(jnp.int32, _LANES)
    for sh in (8, 4, 2, 1):
        idx = lax.bitwise_and(lane + sh, _LANES - 1)
        perm = lax.gather(v, idx[:, None], dimension_numbers=dnums,
                          slice_sizes=(1,),
                          mode=lax.GatherScatterMode.PROMISE_IN_BOUNDS)
        v = v + perm
    return v


def _make_sc_kernel(n_rows, seq, hidden, chunk):
    info = plsc.get_sparse_core_info()
    nc, ns = info.num_cores, info.num_subcores
    nw = nc * ns
    batch = n_rows // seq
    span = seq // nw
    n_chunks = span // chunk
    n_vec = hidden // _LANES

    mesh = plsc.VectorSubcoreMesh(core_axis_name="c", subcore_axis_name="s")

    n_items = n_chunks * batch

    @functools.partial(
        pl.kernel,
        mesh=mesh,
        out_type=jax.ShapeDtypeStruct((n_rows, hidden), jnp.float32),
        scratch_types=[
            pltpu.VMEM((chunk, hidden), jnp.float32),
            pltpu.VMEM((chunk, hidden), jnp.float32),
            pltpu.VMEM((chunk, hidden), jnp.float32),
            pltpu.VMEM((hidden,), jnp.float32),
            pltpu.VMEM((hidden,), jnp.float32),
            pltpu.VMEM((chunk, _LANES), jnp.float32),
            pltpu.VMEM((chunk, _LANES), jnp.float32),
            pltpu.SemaphoreType.DMA,
            pltpu.SemaphoreType.DMA,
            pltpu.SemaphoreType.DMA,
            pltpu.SemaphoreType.DMA,
        ],
    )
    def sc_kernel(in_hbm, pos_hbm, gam_hbm, bet_hbm, out_hbm,
                  x0, x1, pos_v, gam_v, bet_v, mean_v, rs_v,
                  si0, si1, so0, so1):
        wid = lax.axis_index("s") * nc + lax.axis_index("c")
        s0 = wid * span
        xs = (x0, x1)
        sin = (si0, si1)
        sout = (so0, so1)
        pltpu.sync_copy(gam_hbm, gam_v)
        pltpu.sync_copy(bet_hbm, bet_v)

        def seq_base(item):
            return s0 + lax.shift_right_logical(item, 2) * chunk

        def io_row(item):
            return lax.bitwise_and(item, 3) * seq + seq_base(item)

        def in_cp(sl, item):
            return pltpu.make_async_copy(
                in_hbm.at[pl.ds(io_row(item), chunk)], xs[sl], sin[sl])

        def out_cp(sl, item):
            return pltpu.make_async_copy(
                xs[sl], out_hbm.at[pl.ds(io_row(item), chunk)], sout[sl])

        def compute(x_v):
            @plsc.parallel_loop(0, chunk, unroll=4)
            def _(t):
                a1 = [jnp.zeros((_LANES,), jnp.float32) for _ in range(2)]
                a2 = [jnp.zeros((_LANES,), jnp.float32) for _ in range(2)]
                for i in range(n_vec):
                    a = i & 1
                    sl = pl.ds(i * _LANES, _LANES)
                    xv = x_v[t, sl] + pos_v[t, sl]
                    x_v[t, sl] = xv
                    a1[a] = a1[a] + xv
                    a2[a] = a2[a] + xv * xv
                s1 = a1[0] + a1[1]
                s2 = a2[0] + a2[1]
                mean = _splat_sum(s1) * (1.0 / hidden)
                var = _splat_sum(s2) * (1.0 / hidden) - mean * mean
                mean_v[t] = mean
                rs_v[t] = _rsqrt_newton(var + _EPS)

            grp = 16
            for g0 in range(0, n_vec, grp):
                gs = [gam_v[pl.ds(i * _LANES, _LANES)]
                      for i in range(g0, g0 + grp)]
                bs = [bet_v[pl.ds(i * _LANES, _LANES)]
                      for i in range(g0, g0 + grp)]

                @plsc.parallel_loop(0, chunk, unroll=2)
                def _(t, g0=g0, gs=gs, bs=bs):
                    r = rs_v[t]
                    mr = mean_v[t] * r
                    for k in range(grp):
                        sl = pl.ds((g0 + k) * _LANES, _LANES)
                        x_v[t, sl] = (x_v[t, sl] * r - mr) * gs[k] + bs[k]

        in_cp(0, 0).start()
        in_cp(1, 1).start()

        def pair_body(p, _):
            i0 = 2 * p
            for sl in range(2):
                item = i0 + sl
                if sl == 0:
                    @pl.when(lax.bitwise_and(item, 3) == 0)
                    def _():
                        pltpu.sync_copy(
                            pos_hbm.at[pl.ds(seq_base(item), chunk)], pos_v)
                in_cp(sl, item).wait()
                compute(xs[sl])
                out_cp(sl, item).start()
            for sl in range(2):
                item = i0 + sl
                @pl.when(item + 2 < n_items)
                def _():
                    out_cp(sl, item).wait()
                    in_cp(sl, item + 2).start()
            return 0

        lax.fori_loop(0, n_items // 2, pair_body, 0)
        out_cp(0, n_items - 2).wait()
        out_cp(1, n_items - 1).wait()

    return sc_kernel


def kernel(inputs_embeds, position_table, ln_gamma, ln_beta):
    batch, seq, hidden = inputs_embeds.shape
    flat = inputs_embeds.reshape(batch * seq, hidden)
    pos = position_table[:seq]
    fn = _make_sc_kernel(batch * seq, seq, hidden, chunk=32)
    out = fn(flat, pos, ln_gamma, ln_beta)
    return out.reshape(batch, seq, hidden)
